# baseline (speedup 1.0000x reference)
import functools

import jax
import jax.numpy as jnp
from jax import lax
from jax.experimental import pallas as pl
from jax.experimental.pallas import tpu as pltpu
from jax.experimental.pallas import tpu_sc as plsc

VOCAB = 1000000
HIDDEN = 64
BATCH = 16384

NC = 2
NS = 16
NW = NC * NS
BPW = BATCH // NW
CW = 128
CH = BPW // CW


@functools.lru_cache(maxsize=1)
def _sc_gather_build():
    mesh = plsc.VectorSubcoreMesh(core_axis_name="c", subcore_axis_name="s")

    @functools.partial(
        pl.kernel,
        mesh=mesh,
        out_type=jax.ShapeDtypeStruct((BATCH, HIDDEN), jnp.float32),
        compiler_params=pltpu.CompilerParams(use_tc_tiling_on_sc=False),
        scratch_types=[
            pltpu.VMEM((CH, CW), jnp.int32),
            pltpu.VMEM((BPW, HIDDEN), jnp.float32),
            pltpu.SemaphoreType.DMA,
        ],
    )
    def gather_kernel(table_hbm, idx_hbm, out_hbm, idx_v, rows_v, sem):
        wid = lax.axis_index("s") * NC + lax.axis_index("c")
        pltpu.sync_copy(idx_hbm.at[wid], idx_v)
        copies = [
            pltpu.async_copy(
                table_hbm.at[idx_v.at[j]],
                rows_v.at[pl.ds(j * CW, CW)],
                sem,
            )
            for j in range(CH)
        ]
        for c in copies:
            c.wait()
        pltpu.sync_copy(rows_v, out_hbm.at[pl.ds(wid * BPW, BPW)])

    return gather_kernel


_BLK = 2048


def _mlp_body(emb_ref, w1_ref, b1_ref, w2_ref, b2_ref, out_ref):
    h = jnp.dot(emb_ref[...], w1_ref[...], preferred_element_type=jnp.float32)
    h = jnp.maximum(h + b1_ref[...], 0.0)
    logits = jnp.dot(h, w2_ref[...], preferred_element_type=jnp.float32)
    out_ref[...] = jax.nn.sigmoid(logits + b2_ref[...])


def _tc_mlp(emb, W1, b1, W2, b2):
    grid = (BATCH // _BLK,)
    return pl.pallas_call(
        _mlp_body,
        grid=grid,
        in_specs=[
            pl.BlockSpec((_BLK, HIDDEN), lambda i: (i, 0)),
            pl.BlockSpec((HIDDEN, 128), lambda i: (0, 0)),
            pl.BlockSpec((1, 128), lambda i: (0, 0)),
            pl.BlockSpec((128, 1), lambda i: (0, 0)),
            pl.BlockSpec((1, 1), lambda i: (0, 0)),
        ],
        out_specs=pl.BlockSpec((_BLK, 1), lambda i: (i, 0)),
        out_shape=jax.ShapeDtypeStruct((BATCH, 1), jnp.float32),
    )(emb, W1, b1, W2, b2)


def kernel(tool_token, table, W1, b1, W2, b2):
    idx3 = tool_token.astype(jnp.int32).reshape(NW, CH, CW)
    emb = _sc_gather_build()(table, idx3)
    return _tc_mlp(emb, W1, b1.reshape(1, 128), W2, b2.reshape(1, 1))


# speedup vs baseline: 2.3464x; 2.3464x over previous
import functools

import jax
import jax.numpy as jnp
from jax import lax
from jax.experimental import pallas as pl
from jax.experimental.pallas import tpu as pltpu
from jax.experimental.pallas import tpu_sc as plsc

VOCAB = 1000000
HIDDEN = 64
BATCH = 16384

NC = 2
NS = 16
NW = NC * NS
BPW = BATCH // NW
SUB = 8
K = 64
NKC = BPW // K
PR = BPW // 2


@functools.lru_cache(maxsize=1)
def _sc_gather_build():
    mesh = plsc.VectorSubcoreMesh(core_axis_name="c", subcore_axis_name="s")

    @functools.partial(
        pl.kernel,
        mesh=mesh,
        out_type=jax.ShapeDtypeStruct((BATCH // 2, 128), jnp.float32),
        scratch_types=[
            pltpu.VMEM((BPW,), jnp.int32),
            pltpu.VMEM((PR, 128), jnp.float32),
            pltpu.SemaphoreType.DMA,
        ],
    )
    def gather_kernel(table_hbm, tok_hbm, out_hbm, tok_v, rows_v, sem):
        wid = lax.axis_index("s") * NC + lax.axis_index("c")
        pltpu.sync_copy(tok_hbm.at[wid], tok_v)

        def fire_group(g):
            v16 = tok_v[pl.ds(g * 16, 16)]
            for j in range(16):
                t = v16[j]
                pltpu.async_copy(
                    table_hbm.at[t >> 3, t & 7],
                    rows_v.at[g * 8 + (j >> 1), pl.ds((j & 1) * HIDDEN, HIDDEN)],
                    sem,
                )

        def drain_group(g):
            pltpu.make_async_copy(
                out_hbm.at[pl.ds(0, 8)],
                rows_v.at[pl.ds(g * 8, 8)],
                sem,
            ).wait()

        def head(g, _):
            fire_group(g)
            return 0

        def pipelined(g, _):
            fire_group(g)
            drain_group(g - 1)
            return 0

        lax.fori_loop(0, 1, head, 0)
        lax.fori_loop(1, BPW // 16, pipelined, 0)
        drain_group(BPW // 16 - 1)
        pltpu.sync_copy(rows_v, out_hbm.at[pl.ds(wid * PR, PR)])

    return gather_kernel


_BLK = 2048


def _mlp_body(emb_ref, w1_ref, b1_ref, w2_ref, b2_ref, out_ref):
    h = jnp.dot(emb_ref[...], w1_ref[...], preferred_element_type=jnp.float32)
    h = jnp.maximum(h + b1_ref[...], 0.0)
    logits = jnp.dot(h, w2_ref[...], preferred_element_type=jnp.float32)
    out_ref[...] = jax.nn.sigmoid(logits + b2_ref[...])


def _tc_mlp(emb, W1, b1, W2, b2):
    grid = (BATCH // _BLK,)
    return pl.pallas_call(
        _mlp_body,
        grid=grid,
        in_specs=[
            pl.BlockSpec((_BLK, HIDDEN), lambda i: (i, 0)),
            pl.BlockSpec((HIDDEN, 128), lambda i: (0, 0)),
            pl.BlockSpec((1, 128), lambda i: (0, 0)),
            pl.BlockSpec((128, 1), lambda i: (0, 0)),
            pl.BlockSpec((1, 1), lambda i: (0, 0)),
        ],
        out_specs=pl.BlockSpec((_BLK, 1), lambda i: (i, 0)),
        out_shape=jax.ShapeDtypeStruct((BATCH, 1), jnp.float32),
    )(emb, W1, b1, W2, b2)


def kernel(tool_token, table, W1, b1, W2, b2):
    tok2 = tool_token.astype(jnp.int32).reshape(NW, BPW)
    table3 = table.reshape(VOCAB // SUB, SUB, HIDDEN)
    emb2 = _sc_gather_build()(table3, tok2)
    emb = emb2.reshape(BATCH, HIDDEN)
    return _tc_mlp(emb, W1, b1.reshape(1, 128), W2, b2.reshape(1, 1))
